# Initial kernel scaffold; baseline (speedup 1.0000x reference)
import jax
import jax.numpy as jnp
from jax.experimental import pallas as pl


def kernel(x, edge_index, lin1_W, lin1_b, conv1_W, conv1_phiW, conv1_b, lin2_W, lin2_b, conv2_W, conv2_phiW, conv2_b):
    raise NotImplementedError("write your pallas kernel here")


# speedup vs baseline: 2.9887x; 2.9887x over previous
import functools

import jax
import jax.numpy as jnp
from jax import lax
from jax.experimental import pallas as pl
from jax.experimental.pallas import tpu as pltpu

N_NODES = 10000
D_FEAT = 128
HIDDEN = 128
N_CLASSES = 40
C_PAD = 48
NACC = 10240
ROWS = 2000
GAMMA = 0.1
EPS = 0.1

_HI = lax.Precision.HIGHEST


def _dot(a, b):
    return lax.dot_general(a, b, (((1,), (0,)), ((), ())), precision=_HI,
                           preferred_element_type=jnp.float32)


def _stage_a_body(x_ref, dis_ref, w1t_ref, b1_ref, phi1t_ref, h1_ref, y1_ref):
    h1 = jnp.maximum(_dot(x_ref[...], w1t_ref[...]) + b1_ref[...], 0.0)
    h1_ref[...] = h1
    y1_ref[...] = dis_ref[...] * _dot(h1, phi1t_ref[...])


def _stage_a(x, dis_col, w1t, b1, phi1t):
    grid = (N_NODES // ROWS,)
    return pl.pallas_call(
        _stage_a_body,
        grid=grid,
        in_specs=[
            pl.BlockSpec((ROWS, D_FEAT), lambda i: (i, 0)),
            pl.BlockSpec((ROWS, 1), lambda i: (i, 0)),
            pl.BlockSpec((D_FEAT, HIDDEN), lambda i: (0, 0)),
            pl.BlockSpec((1, HIDDEN), lambda i: (0, 0)),
            pl.BlockSpec((HIDDEN, HIDDEN), lambda i: (0, 0)),
        ],
        out_specs=[
            pl.BlockSpec((ROWS, HIDDEN), lambda i: (i, 0)),
            pl.BlockSpec((ROWS, HIDDEN), lambda i: (i, 0)),
        ],
        out_shape=[
            jax.ShapeDtypeStruct((N_NODES, HIDDEN), jnp.float32),
            jax.ShapeDtypeStruct((N_NODES, HIDDEN), jnp.float32),
        ],
    )(x, dis_col, w1t, b1, phi1t)


def _stage_b_body(h1_ref, y1_ref, dis_ref, p0_ref, p1_ref,
                  aw1t_ref, b1c_ref, w2t_ref, b2_ref, phi2t_ref,
                  h3_ref, y2p_ref):
    h1 = h1_ref[...]
    scat = p0_ref[0] + p1_ref[0]
    gcn = dis_ref[...] * (scat + y1_ref[...])
    h = jnp.tanh(_dot(h1, aw1t_ref[...]) + gcn + b1c_ref[...])
    h2 = h1 + EPS * h
    h3 = _dot(h2, w2t_ref[...]) + b2_ref[...]
    h3_ref[...] = h3
    y2 = dis_ref[...] * _dot(h3, phi2t_ref[...])
    y2p_ref[...] = jnp.concatenate(
        [y2, jnp.zeros((ROWS, C_PAD - N_CLASSES), jnp.float32)], axis=1)


def _stage_b(h1, y1, dis_col, scat1_p, aw1t, b1c, w2t, b2, phi2t):
    grid = (N_NODES // ROWS,)
    return pl.pallas_call(
        _stage_b_body,
        grid=grid,
        in_specs=[
            pl.BlockSpec((ROWS, HIDDEN), lambda i: (i, 0)),
            pl.BlockSpec((ROWS, HIDDEN), lambda i: (i, 0)),
            pl.BlockSpec((ROWS, 1), lambda i: (i, 0)),
            pl.BlockSpec((1, ROWS, HIDDEN), lambda i: (0, i, 0)),
            pl.BlockSpec((1, ROWS, HIDDEN), lambda i: (1, i, 0)),
            pl.BlockSpec((HIDDEN, HIDDEN), lambda i: (0, 0)),
            pl.BlockSpec((1, HIDDEN), lambda i: (0, 0)),
            pl.BlockSpec((HIDDEN, N_CLASSES), lambda i: (0, 0)),
            pl.BlockSpec((1, N_CLASSES), lambda i: (0, 0)),
            pl.BlockSpec((N_CLASSES, N_CLASSES), lambda i: (0, 0)),
        ],
        out_specs=[
            pl.BlockSpec((ROWS, N_CLASSES), lambda i: (i, 0)),
            pl.BlockSpec((ROWS, C_PAD), lambda i: (i, 0)),
        ],
        out_shape=[
            jax.ShapeDtypeStruct((N_NODES, N_CLASSES), jnp.float32),
            jax.ShapeDtypeStruct((N_NODES, C_PAD), jnp.float32),
        ],
    )(h1, y1, dis_col, scat1_p, scat1_p, aw1t, b1c, w2t, b2, phi2t)


def _stage_c_body(h3_ref, y2p_ref, dis_ref, q0_ref, q1_ref,
                  aw2t_ref, b2c_ref, out_ref):
    h3 = h3_ref[...]
    scat = (q0_ref[0] + q1_ref[0])[:, :N_CLASSES]
    gcn = dis_ref[...] * (scat + y2p_ref[:, :N_CLASSES])
    h = jnp.tanh(_dot(h3, aw2t_ref[...]) + gcn + b2c_ref[...])
    h4 = h3 + EPS * h
    m = jnp.max(h4, axis=1, keepdims=True)
    lse = jnp.log(jnp.sum(jnp.exp(h4 - m), axis=1, keepdims=True))
    out_ref[...] = h4 - m - lse


def _stage_c(h3, y2p, dis_col, scat2_p, aw2t, b2c):
    grid = (N_NODES // ROWS,)
    return pl.pallas_call(
        _stage_c_body,
        grid=grid,
        in_specs=[
            pl.BlockSpec((ROWS, N_CLASSES), lambda i: (i, 0)),
            pl.BlockSpec((ROWS, C_PAD), lambda i: (i, 0)),
            pl.BlockSpec((ROWS, 1), lambda i: (i, 0)),
            pl.BlockSpec((1, ROWS, C_PAD), lambda i: (0, i, 0)),
            pl.BlockSpec((1, ROWS, C_PAD), lambda i: (1, i, 0)),
            pl.BlockSpec((N_CLASSES, N_CLASSES), lambda i: (0, 0)),
            pl.BlockSpec((1, N_CLASSES), lambda i: (0, 0)),
        ],
        out_specs=pl.BlockSpec((ROWS, N_CLASSES), lambda i: (i, 0)),
        out_shape=jax.ShapeDtypeStruct((N_NODES, N_CLASSES), jnp.float32),
    )(h3, y2p, dis_col, scat2_p, scat2_p, aw2t, b2c)


def _deg_partials(dst):
    p0 = jnp.zeros((NACC,), jnp.float32).at[dst].add(1.0)
    return jnp.stack([p0, jnp.zeros((NACC,), jnp.float32)])


def _scatter_partials(y, src, dst, d):
    p0 = jnp.zeros((NACC, d), jnp.float32).at[dst].add(y[src])
    return jnp.stack([p0, jnp.zeros((NACC, d), jnp.float32)])


def kernel(x, edge_index, lin1_W, lin1_b, conv1_W, conv1_phiW, conv1_b,
           lin2_W, lin2_b, conv2_W, conv2_phiW, conv2_b):
    src = edge_index[0]
    dst = edge_index[1]

    w1t = lin1_W.T
    b1 = lin1_b[None, :]
    phi1t = conv1_phiW.T
    aw1t = conv1_W.T - conv1_W - GAMMA * jnp.eye(HIDDEN, dtype=jnp.float32)
    b1c = conv1_b[None, :]
    w2t = lin2_W.T
    b2 = lin2_b[None, :]
    phi2t = conv2_phiW.T
    aw2t = conv2_W.T - conv2_W - GAMMA * jnp.eye(N_CLASSES, dtype=jnp.float32)
    b2c = conv2_b[None, :]

    deg_p = _deg_partials(dst)
    dis_col = lax.rsqrt(deg_p[0, :N_NODES] + deg_p[1, :N_NODES] + 1.0)[:, None]

    h1, y1 = _stage_a(x, dis_col, w1t, b1, phi1t)
    scat1_p = _scatter_partials(y1, src, dst, HIDDEN)
    h3, y2p = _stage_b(h1, y1, dis_col, scat1_p, aw1t, b1c, w2t, b2, phi2t)
    scat2_p = _scatter_partials(y2p, src, dst, C_PAD)
    return _stage_c(h3, y2p, dis_col, scat2_p, aw2t, b2c)


# speedup vs baseline: 15.9865x; 5.3489x over previous
import functools

import jax
import jax.numpy as jnp
from jax import lax
from jax.experimental import pallas as pl
from jax.experimental.pallas import tpu as pltpu
from jax.experimental.pallas import tpu_sc as plsc

N_NODES = 10000
D_FEAT = 128
HIDDEN = 128
N_CLASSES = 40
C_PAD = 48
NACC = 10240
ROWS = 2000
GAMMA = 0.1
EPS = 0.1

NC = 2
NS = 16
NW = NC * NS
K_EDGE = 128
NB_EDGE = 79
EPT = NB_EDGE * K_EDGE
E_PAD = NW * EPT
RPT = NACC // NS
ZR = 64

_HI = lax.Precision.HIGHEST


def _dot(a, b):
    return lax.dot_general(a, b, (((1,), (0,)), ((), ())), precision=_HI,
                           preferred_element_type=jnp.float32)


def _stage_a_body(x_ref, dis_ref, w1t_ref, b1_ref, phi1t_ref, h1_ref, y1_ref):
    h1 = jnp.maximum(_dot(x_ref[...], w1t_ref[...]) + b1_ref[...], 0.0)
    h1_ref[...] = h1
    y1_ref[...] = dis_ref[...] * _dot(h1, phi1t_ref[...])


def _stage_a(x, dis_col, w1t, b1, phi1t):
    grid = (N_NODES // ROWS,)
    return pl.pallas_call(
        _stage_a_body,
        grid=grid,
        in_specs=[
            pl.BlockSpec((ROWS, D_FEAT), lambda i: (i, 0)),
            pl.BlockSpec((ROWS, 1), lambda i: (i, 0)),
            pl.BlockSpec((D_FEAT, HIDDEN), lambda i: (0, 0)),
            pl.BlockSpec((1, HIDDEN), lambda i: (0, 0)),
            pl.BlockSpec((HIDDEN, HIDDEN), lambda i: (0, 0)),
        ],
        out_specs=[
            pl.BlockSpec((ROWS, HIDDEN), lambda i: (i, 0)),
            pl.BlockSpec((ROWS, HIDDEN), lambda i: (i, 0)),
        ],
        out_shape=[
            jax.ShapeDtypeStruct((N_NODES, HIDDEN), jnp.float32),
            jax.ShapeDtypeStruct((N_NODES, HIDDEN), jnp.float32),
        ],
    )(x, dis_col, w1t, b1, phi1t)


def _stage_b_body(h1_ref, y1_ref, dis_ref, p0_ref, p1_ref,
                  aw1t_ref, b1c_ref, w2t_ref, b2_ref, phi2t_ref,
                  h3_ref, y2p_ref):
    h1 = h1_ref[...]
    scat = p0_ref[0] + p1_ref[0]
    gcn = dis_ref[...] * (scat + y1_ref[...])
    h = jnp.tanh(_dot(h1, aw1t_ref[...]) + gcn + b1c_ref[...])
    h2 = h1 + EPS * h
    h3 = _dot(h2, w2t_ref[...]) + b2_ref[...]
    h3_ref[...] = h3
    y2 = dis_ref[...] * _dot(h3, phi2t_ref[...])
    y2p_ref[...] = jnp.concatenate(
        [y2, jnp.zeros((ROWS, C_PAD - N_CLASSES), jnp.float32)], axis=1)


def _stage_b(h1, y1, dis_col, scat1_p, aw1t, b1c, w2t, b2, phi2t):
    grid = (N_NODES // ROWS,)
    return pl.pallas_call(
        _stage_b_body,
        grid=grid,
        in_specs=[
            pl.BlockSpec((ROWS, HIDDEN), lambda i: (i, 0)),
            pl.BlockSpec((ROWS, HIDDEN), lambda i: (i, 0)),
            pl.BlockSpec((ROWS, 1), lambda i: (i, 0)),
            pl.BlockSpec((1, ROWS, HIDDEN), lambda i: (0, i, 0)),
            pl.BlockSpec((1, ROWS, HIDDEN), lambda i: (1, i, 0)),
            pl.BlockSpec((HIDDEN, HIDDEN), lambda i: (0, 0)),
            pl.BlockSpec((1, HIDDEN), lambda i: (0, 0)),
            pl.BlockSpec((HIDDEN, N_CLASSES), lambda i: (0, 0)),
            pl.BlockSpec((1, N_CLASSES), lambda i: (0, 0)),
            pl.BlockSpec((N_CLASSES, N_CLASSES), lambda i: (0, 0)),
        ],
        out_specs=[
            pl.BlockSpec((ROWS, N_CLASSES), lambda i: (i, 0)),
            pl.BlockSpec((ROWS, C_PAD), lambda i: (i, 0)),
        ],
        out_shape=[
            jax.ShapeDtypeStruct((N_NODES, N_CLASSES), jnp.float32),
            jax.ShapeDtypeStruct((N_NODES, C_PAD), jnp.float32),
        ],
    )(h1, y1, dis_col, scat1_p, scat1_p, aw1t, b1c, w2t, b2, phi2t)


def _stage_c_body(h3_ref, y2p_ref, dis_ref, q0_ref, q1_ref,
                  aw2t_ref, b2c_ref, out_ref):
    h3 = h3_ref[...]
    scat = (q0_ref[0] + q1_ref[0])[:, :N_CLASSES]
    gcn = dis_ref[...] * (scat + y2p_ref[:, :N_CLASSES])
    h = jnp.tanh(_dot(h3, aw2t_ref[...]) + gcn + b2c_ref[...])
    h4 = h3 + EPS * h
    m = jnp.max(h4, axis=1, keepdims=True)
    lse = jnp.log(jnp.sum(jnp.exp(h4 - m), axis=1, keepdims=True))
    out_ref[...] = h4 - m - lse


def _stage_c(h3, y2p, dis_col, scat2_p, aw2t, b2c):
    grid = (N_NODES // ROWS,)
    return pl.pallas_call(
        _stage_c_body,
        grid=grid,
        in_specs=[
            pl.BlockSpec((ROWS, N_CLASSES), lambda i: (i, 0)),
            pl.BlockSpec((ROWS, C_PAD), lambda i: (i, 0)),
            pl.BlockSpec((ROWS, 1), lambda i: (i, 0)),
            pl.BlockSpec((1, ROWS, C_PAD), lambda i: (0, i, 0)),
            pl.BlockSpec((1, ROWS, C_PAD), lambda i: (1, i, 0)),
            pl.BlockSpec((N_CLASSES, N_CLASSES), lambda i: (0, 0)),
            pl.BlockSpec((1, N_CLASSES), lambda i: (0, 0)),
        ],
        out_specs=pl.BlockSpec((ROWS, N_CLASSES), lambda i: (i, 0)),
        out_shape=jax.ShapeDtypeStruct((N_NODES, N_CLASSES), jnp.float32),
    )(h3, y2p, dis_col, scat2_p, scat2_p, aw2t, b2c)


def _sc_mesh():
    return plsc.VectorSubcoreMesh(core_axis_name="c", subcore_axis_name="s")


DEG_R = NACC // 128


def _deg_body(dst_hbm, out_hbm, dst_v, acc_v, rowidx_v, acc_sh):
    c = lax.axis_index("c")
    s = lax.axis_index("s")
    wid = c * NS + s
    z16 = jnp.zeros((16,), jnp.float32)
    iota16 = lax.iota(jnp.int32, 16)

    def zero(i, carry):
        acc_v[i // 8, pl.ds((i % 8) * 16, 16)] = z16
        return carry
    lax.fori_loop(0, DEG_R * 8, zero, 0)

    def mkidx(i, carry):
        rowidx_v[pl.ds(i * 16, 16)] = iota16 + i * 16
        return carry
    lax.fori_loop(0, DEG_R // 16, mkidx, 0)

    pltpu.sync_copy(dst_hbm.at[wid], dst_v)

    @pl.when(s == 0)
    def _():
        pltpu.sync_copy(acc_v, acc_sh)
    plsc.subcore_barrier()

    ones16 = jnp.ones((16,), jnp.float32)

    def step(i, carry):
        idx = dst_v[pl.ds(i * 16, 16)]
        plsc.addupdate_scatter(
            acc_v, [lax.shift_right_logical(idx, 7),
                    lax.bitwise_and(idx, 127)], ones16)
        return carry
    lax.fori_loop(0, EPT // 16, step, 0)

    pltpu.sync_copy(acc_v, acc_sh.at[rowidx_v], add=True)
    plsc.subcore_barrier()

    @pl.when(s == 0)
    def _():
        pltpu.sync_copy(acc_sh, out_hbm.at[c])


def _deg_partials(dst_flat):
    k = functools.partial(
        pl.kernel,
        out_type=jax.ShapeDtypeStruct((NC, DEG_R, 128), jnp.float32),
        mesh=_sc_mesh(),
        compiler_params=pltpu.CompilerParams(needs_layout_passes=False),
        scratch_types=[
            pltpu.VMEM((EPT,), jnp.int32),
            pltpu.VMEM((DEG_R, 128), jnp.float32),
            pltpu.VMEM((DEG_R,), jnp.int32),
            pltpu.VMEM_SHARED((DEG_R, 128), jnp.float32),
        ],
    )(_deg_body)
    return k(dst_flat)


def _scatter_body(d, y_hbm, src_hbm, dst_hbm, out_hbm,
                  src_v, dst_v, rows_v, zb, acc_sh, sem):
    c = lax.axis_index("c")
    s = lax.axis_index("s")
    wid = c * NS + s
    z16 = jnp.zeros((16,), jnp.float32)
    vpr = d // 16

    def zero(i, carry):
        zb[i // vpr, pl.ds((i % vpr) * 16, 16)] = z16
        return carry
    lax.fori_loop(0, ZR * vpr, zero, 0)

    def zfill(i, carry):
        pltpu.sync_copy(zb, acc_sh.at[pl.ds(s * RPT + i * ZR, ZR)])
        return carry
    lax.fori_loop(0, RPT // ZR, zfill, 0)

    pltpu.sync_copy(src_hbm.at[wid], src_v)
    pltpu.sync_copy(dst_hbm.at[wid], dst_v)
    plsc.subcore_barrier()

    def step(j, carry):
        pltpu.async_copy(y_hbm.at[src_v.at[j]], rows_v, sem).wait()
        pltpu.sync_copy(rows_v, acc_sh.at[dst_v.at[j]], add=True)
        return carry
    lax.fori_loop(0, NB_EDGE, step, 0)

    plsc.subcore_barrier()

    def out(i, carry):
        pltpu.sync_copy(acc_sh.at[pl.ds(s * RPT + i * ZR, ZR)],
                        out_hbm.at[c, pl.ds(s * RPT + i * ZR, ZR)])
        return carry
    lax.fori_loop(0, RPT // ZR, out, 0)


def _scatter_partials(y, src3, dst3, d):
    k = functools.partial(
        pl.kernel,
        out_type=jax.ShapeDtypeStruct((NC, NACC, d), jnp.float32),
        mesh=_sc_mesh(),
        compiler_params=pltpu.CompilerParams(use_tc_tiling_on_sc=False),
        scratch_types=[
            pltpu.VMEM((NB_EDGE, K_EDGE), jnp.int32),
            pltpu.VMEM((NB_EDGE, K_EDGE), jnp.int32),
            pltpu.VMEM((K_EDGE, d), jnp.float32),
            pltpu.VMEM((ZR, d), jnp.float32),
            pltpu.VMEM_SHARED((NACC, d), jnp.float32),
            pltpu.SemaphoreType.DMA,
        ],
    )(functools.partial(_scatter_body, d))
    return k(y, src3, dst3)


def kernel(x, edge_index, lin1_W, lin1_b, conv1_W, conv1_phiW, conv1_b,
           lin2_W, lin2_b, conv2_W, conv2_phiW, conv2_b):
    src = edge_index[0]
    dst = edge_index[1]

    w1t = lin1_W.T
    b1 = lin1_b[None, :]
    phi1t = conv1_phiW.T
    aw1t = conv1_W.T - conv1_W - GAMMA * jnp.eye(HIDDEN, dtype=jnp.float32)
    b1c = conv1_b[None, :]
    w2t = lin2_W.T
    b2 = lin2_b[None, :]
    phi2t = conv2_phiW.T
    aw2t = conv2_W.T - conv2_W - GAMMA * jnp.eye(N_CLASSES, dtype=jnp.float32)
    b2c = conv2_b[None, :]

    pad = E_PAD - src.shape[0]
    srcp = jnp.concatenate([src.astype(jnp.int32),
                            jnp.zeros((pad,), jnp.int32)])
    dstp = jnp.concatenate([dst.astype(jnp.int32),
                            jnp.full((pad,), N_NODES, jnp.int32)])
    src3 = srcp.reshape(NW, NB_EDGE, K_EDGE)
    dst3 = dstp.reshape(NW, NB_EDGE, K_EDGE)
    dst_flat = dstp.reshape(NW, EPT)

    deg_p = _deg_partials(dst_flat).reshape(NC, NACC)
    dis_col = lax.rsqrt(deg_p[0, :N_NODES] + deg_p[1, :N_NODES] + 1.0)[:, None]

    h1, y1 = _stage_a(x, dis_col, w1t, b1, phi1t)
    scat1_p = _scatter_partials(y1, src3, dst3, HIDDEN)
    h3, y2p = _stage_b(h1, y1, dis_col, scat1_p, aw1t, b1c, w2t, b2, phi2t)
    scat2_p = _scatter_partials(y2p, src3, dst3, C_PAD)
    return _stage_c(h3, y2p, dis_col, scat2_p, aw2t, b2c)
